# baseline (speedup 1.0000x reference)
import functools

import jax
import jax.numpy as jnp
import numpy as np
from jax import lax
from jax.experimental import pallas as pl
from jax.experimental.pallas import tpu as pltpu
from jax.experimental.pallas import tpu_sc as plsc

N = 10000
E = 320000
D = 128
NPAD = 10240
W = 128
NWIN = E // W
NC = 2
NS = 16
NWK = NC * NS
ZR = NPAD // NS

NCHUNK = 4
EC = E // NCHUNK
NWINC = NWIN // NCHUNK
NJC = NWINC // NWK + 1

_mesh = plsc.VectorSubcoreMesh(core_axis_name="c", subcore_axis_name="s")


def _wrange(wid):
    w0 = wid * (NWINC // NWK) + jnp.minimum(wid, NWINC % NWK)
    cnt = NWINC // NWK + (wid < NWINC % NWK).astype(jnp.int32)
    return w0, cnt


def _pre(x, w1a, w1b, na, eb1, nb1):
    def body(x_r, wa_r, wb_r, na_r, eb1_r, nb1_r, xa_r, xb_r, xn_r):
        xv = x_r[...]
        xa_r[...] = jnp.dot(xv, wa_r[...], preferred_element_type=jnp.float32) + eb1_r[...]
        xb_r[...] = jnp.dot(xv, wb_r[...], preferred_element_type=jnp.float32)
        xn_r[...] = jnp.dot(xv, na_r[...], preferred_element_type=jnp.float32) + nb1_r[...]

    NB = 2000
    blk = pl.BlockSpec((NB, D), lambda i: (i, 0))
    wblk = pl.BlockSpec((D, D), lambda i: (0, 0))
    bblk = pl.BlockSpec((1, D), lambda i: (0, 0))
    outf = jax.ShapeDtypeStruct((N, D), jnp.float32)
    return pl.pallas_call(
        body,
        grid=(N // NB,),
        in_specs=[blk, wblk, wblk, wblk, bblk, bblk],
        out_specs=(blk, blk, blk),
        out_shape=(outf, outf, outf),
    )(x, w1a, w1b, na, eb1, nb1)


def _gather(xa, xb, row4, col4, idmat, c):
    @functools.partial(
        pl.kernel,
        out_type=jax.ShapeDtypeStruct((EC, D), jnp.float32),
        mesh=_mesh,
        scratch_types=[
            pltpu.VMEM((NJC, W), jnp.int32),
            pltpu.VMEM((NJC, W), jnp.int32),
            pltpu.VMEM((1, W), jnp.int32),
            pltpu.VMEM((W, D), jnp.float32),
            pltpu.VMEM((W, D), jnp.float32),
            pltpu.VMEM((W, D), jnp.float32),
            pltpu.VMEM((W, D), jnp.float32),
            pltpu.VMEM_SHARED((NS * W, D), jnp.float32),
            pltpu.VMEM_SHARED((NS * W, D), jnp.float32),
            pltpu.SemaphoreType.DMA,
            pltpu.SemaphoreType.DMA,
            pltpu.SemaphoreType.DMA,
            pltpu.SemaphoreType.DMA,
        ],
    )
    def k(xa_hbm, xb_hbm, row_hbm, col_hbm, idm_hbm, g_hbm,
          ridx, cidx, ida, r1a, r2a, r1b, r2b, spa, spb, sa, sb, swa, swb):
        cid = lax.axis_index("c")
        sid = lax.axis_index("s")
        wid = sid * NC + cid
        w0, cnt = _wrange(wid)
        my0 = sid * W

        pltpu.sync_copy(row_hbm.at[c, wid], ridx)
        pltpu.sync_copy(col_hbm.at[c, wid], cidx)
        pltpu.sync_copy(idm_hbm.at[sid], ida)

        pltpu.async_copy(xa_hbm.at[ridx.at[0]], r1a, sa)
        pltpu.async_copy(xb_hbm.at[cidx.at[0]], r2a, sa)

        def slot(k_, my1, my2, mysem, ot1, ot2, otsem, mysp, mywsem):
            @pl.when(k_ < cnt)
            def _():
                pltpu.make_async_copy(xa_hbm.at[ridx.at[0]], my1, mysem).wait()
                pltpu.make_async_copy(xb_hbm.at[cidx.at[0]], my2, mysem).wait()

                @pl.when(k_ + 1 < cnt)
                def _():
                    pltpu.async_copy(xa_hbm.at[ridx.at[k_ + 1]], ot1, otsem)
                    pltpu.async_copy(xb_hbm.at[cidx.at[k_ + 1]], ot2, otsem)

                @pl.when(k_ >= 2)
                def _():
                    pltpu.make_async_copy(mysp.at[pl.ds(my0, W)],
                                          g_hbm.at[pl.ds(0, W)], mywsem).wait()

                pltpu.sync_copy(my1, mysp.at[pl.ds(my0, W)])
                pltpu.sync_copy(my2, mysp.at[ida.at[0]], add=True)
                pltpu.async_copy(mysp.at[pl.ds(my0, W)],
                                 g_hbm.at[pl.ds((w0 + k_) * W, W)], mywsem)

        @pl.loop(0, NJC + 1, step=2)
        def _(k_):
            slot(k_, r1a, r2a, sa, r1b, r2b, sb, spa, swa)
            slot(k_ + 1, r1b, r2b, sb, r1a, r2a, sa, spb, swb)

        pltpu.make_async_copy(spa.at[pl.ds(my0, W)],
                              g_hbm.at[pl.ds(0, W)], swa).wait()
        pltpu.make_async_copy(spb.at[pl.ds(my0, W)],
                              g_hbm.at[pl.ds(0, W)], swb).wait()

    return k(xa, xb, row4, col4, idmat)


def _edge(g, ea_full, w1c, w2, eb2, c):
    EB = 3200
    NBLK = EC // EB

    def body(g_r, ea_r, w1_r, w2_r, b2_r, out_r):
        eav = ea_r[...]
        cc = jnp.dot(eav.astype(jnp.bfloat16), w1_r[...], preferred_element_type=jnp.float32)
        h = jnp.maximum(g_r[...] + cc, 0.0)
        out_r[...] = (eav + jnp.dot(h.astype(jnp.bfloat16), w2_r[...],
                                    preferred_element_type=jnp.float32) + b2_r[...])

    blk = pl.BlockSpec((EB, D), lambda i: (i, 0))
    eablk = pl.BlockSpec((EB, D), lambda i, c=c: (c * NBLK + i, 0))
    wblk = pl.BlockSpec((D, D), lambda i: (0, 0))
    bblk = pl.BlockSpec((1, D), lambda i: (0, 0))
    return pl.pallas_call(
        body,
        grid=(NBLK,),
        in_specs=[blk, eablk, wblk, wblk, bblk],
        out_specs=blk,
        out_shape=jax.ShapeDtypeStruct((EC, D), jnp.float32),
    )(g, ea_full, w1c, w2, eb2)


NJS = NWINC // 8 + 1


def _swrange(u):
    lw0 = u * (NWINC // 8) + jnp.minimum(u, NWINC % 8)
    lcnt = NWINC // 8 + (u < NWINC % 8).astype(jnp.int32)
    return lw0, lcnt


def _scatter(eouts, col3, zrows):
    @functools.partial(
        pl.kernel,
        out_type=(jax.ShapeDtypeStruct((NC, NPAD, D), jnp.float32),
                  jax.ShapeDtypeStruct((E, D), jnp.float32)),
        mesh=_mesh,
        scratch_types=[
            pltpu.VMEM((NJS, W), jnp.int32),
            pltpu.VMEM((W, D), jnp.float32),
            pltpu.VMEM((W, D), jnp.float32),
            pltpu.VMEM_SHARED((NPAD, D), jnp.float32),
            pltpu.SemaphoreType.DMA,
            pltpu.SemaphoreType.DMA,
            pltpu.SemaphoreType.DMA,
            pltpu.SemaphoreType.DMA,
        ],
    )
    def k(e0_hbm, e1_hbm, e2_hbm, e3_hbm, col_hbm, z_hbm, out_hbm, eo_hbm,
          cidx, ebufa, ebufb, acc, sa, sb, swa, swb):
        cid = lax.axis_index("c")
        sid = lax.axis_index("s")
        wid = sid * NC + cid
        u = lax.rem(wid, 8)
        lw0, lcnt = _swrange(u)

        pltpu.sync_copy(z_hbm, acc.at[pl.ds(sid * ZR, ZR)])
        pltpu.sync_copy(col_hbm.at[wid], cidx)
        plsc.subcore_barrier()

        def chunk_loop(e_hbm, c):
            pltpu.async_copy(e_hbm.at[pl.ds(lw0 * W, W)], ebufa, sa)

            def slot(k_, mybuf, mysem, otbuf, otsem, otwsem, mywsem):
                @pl.when(k_ < lcnt)
                def _():
                    pltpu.make_async_copy(e_hbm.at[pl.ds(0, W)], mybuf,
                                          mysem).wait()

                    @pl.when(k_ + 1 < lcnt)
                    def _():
                        @pl.when(k_ >= 1)
                        def _():
                            pltpu.make_async_copy(
                                otbuf, eo_hbm.at[pl.ds(0, W)], otwsem).wait()

                        pltpu.async_copy(e_hbm.at[pl.ds((lw0 + k_ + 1) * W, W)],
                                         otbuf, otsem)

                    pltpu.sync_copy(mybuf, acc.at[cidx.at[k_]], add=True)
                    pltpu.async_copy(
                        mybuf, eo_hbm.at[pl.ds(c * EC + (lw0 + k_) * W, W)],
                        mywsem)

            @pl.loop(0, NJS + 1, step=2)
            def _(k_):
                slot(k_, ebufa, sa, ebufb, sb, swb, swa)
                slot(k_ + 1, ebufb, sb, ebufa, sa, swa, swb)

            pltpu.make_async_copy(ebufa, eo_hbm.at[pl.ds(0, W)], swa).wait()
            pltpu.make_async_copy(ebufb, eo_hbm.at[pl.ds(0, W)], swb).wait()

        for c, e_hbm in enumerate((e0_hbm, e1_hbm, e2_hbm, e3_hbm)):
            @pl.when(jnp.logical_and(wid >= 8 * c, wid < 8 * (c + 1)))
            def _(e_hbm=e_hbm, c=c):
                chunk_loop(e_hbm, c)

        plsc.subcore_barrier()
        pltpu.sync_copy(acc.at[pl.ds(sid * ZR, ZR)],
                        out_hbm.at[cid, pl.ds(sid * ZR, ZR)])

    return k(*eouts, col3, zrows)


def _node(x, xn, partials, nbw, nw2, nb2):
    NB = 2000

    def body(x_r, xn_r, p_r, nb_r, w2_r, b2_r, out_r):
        agg = p_r[0] + p_r[1]
        h2 = jnp.maximum(xn_r[...] + jnp.dot(agg, nb_r[...], preferred_element_type=jnp.float32), 0.0)
        out_r[...] = x_r[...] + jnp.dot(h2, w2_r[...], preferred_element_type=jnp.float32) + b2_r[...]

    blk = pl.BlockSpec((NB, D), lambda i: (i, 0))
    pblk = pl.BlockSpec((NC, NB, D), lambda i: (0, i, 0))
    wblk = pl.BlockSpec((D, D), lambda i: (0, 0))
    bblk = pl.BlockSpec((1, D), lambda i: (0, 0))
    return pl.pallas_call(
        body,
        grid=(N // NB,),
        in_specs=[blk, blk, pblk, wblk, wblk, bblk],
        out_specs=blk,
        out_shape=jax.ShapeDtypeStruct((N, D), jnp.float32),
    )(x, xn, partials, nbw, nw2, nb2)


def kernel(x, edge_attr, edge_index, eW1, eb1, eW2, eb2, nW1, nb1, nW2, nb2):
    w1a, w1b, w1c = eW1[:D], eW1[D:2 * D], eW1[2 * D:]
    na, nbw = nW1[:D], nW1[D:]
    eb1r = eb1.reshape(1, D)
    eb2r = eb2.reshape(1, D)
    nb1r = nb1.reshape(1, D)
    nb2r = nb2.reshape(1, D)

    ei = edge_index.reshape(-1).astype(jnp.int32)

    def wflat(w, j):
        return w * W + j if w < NWIN else 0

    gwin = [c * NWINC + t * (NWINC // NWK) + min(t, NWINC % NWK) + k
            for c in range(NCHUNK) for t in range(NWK) for k in range(NJC)]
    gidx = np.array([[wflat(w, j) for j in range(W)] for w in gwin],
                    dtype=np.int32)
    swin = [(t // 8) * NWINC + (t % 8) * (NWINC // 8) + min(t % 8, NWINC % 8) + k
            for t in range(NWK) for k in range(NJS)]
    sidx = np.array([[wflat(w, j) for j in range(W)] for w in swin],
                    dtype=np.int32)
    row4 = jnp.take(ei, gidx.reshape(-1)).reshape(NCHUNK, NWK, NJC, W)
    col4 = jnp.take(ei, (gidx + E).reshape(-1)).reshape(NCHUNK, NWK, NJC, W)
    col3s = jnp.take(ei, (sidx + E).reshape(-1)).reshape(NWK, NJS, W)

    idmat = (jnp.arange(NS, dtype=jnp.int32)[:, None, None] * W
             + jnp.arange(W, dtype=jnp.int32)[None, None, :])
    zrows = jnp.zeros((ZR, D), jnp.float32)
    w1cb = w1c.astype(jnp.bfloat16)
    w2b = eW2.astype(jnp.bfloat16)

    xa, xb, xn = _pre(x, w1a, w1b, na, eb1r, nb1r)

    eouts = []
    for c in range(NCHUNK):
        g_c = _gather(xa, xb, row4, col4, idmat, c)
        eouts.append(_edge(g_c, edge_attr, w1cb, w2b, eb2r, c))

    partials, eout = _scatter(eouts, col3s, zrows)
    x_new = _node(x, xn, partials, nbw, nW2, nb2r)
    return (x_new, eout)


# speedup vs baseline: 1.1157x; 1.1157x over previous
import functools

import jax
import jax.numpy as jnp
import numpy as np
from jax import lax
from jax.experimental import pallas as pl
from jax.experimental.pallas import tpu as pltpu
from jax.experimental.pallas import tpu_sc as plsc

N = 10000
E = 320000
D = 128
NPAD = 10240
W = 128
NWIN = E // W
NC = 2
NS = 16
NWK = NC * NS
ZR = NPAD // NS

NCHUNK = 4
EC = E // NCHUNK
NWINC = NWIN // NCHUNK
NJC = NWINC // NWK + 1

_mesh = plsc.VectorSubcoreMesh(core_axis_name="c", subcore_axis_name="s")


def _wrange(wid):
    w0 = wid * (NWINC // NWK) + jnp.minimum(wid, NWINC % NWK)
    cnt = NWINC // NWK + (wid < NWINC % NWK).astype(jnp.int32)
    return w0, cnt


def _pre(x, w1a, w1b, na, eb1, nb1):
    def body(x_r, wa_r, wb_r, na_r, eb1_r, nb1_r, xa_r, xb_r, xn_r):
        xv = x_r[...]
        xa_r[...] = jnp.dot(xv, wa_r[...], preferred_element_type=jnp.float32) + eb1_r[...]
        xb_r[...] = jnp.dot(xv, wb_r[...], preferred_element_type=jnp.float32)
        xn_r[...] = jnp.dot(xv, na_r[...], preferred_element_type=jnp.float32) + nb1_r[...]

    NB = 2000
    blk = pl.BlockSpec((NB, D), lambda i: (i, 0))
    wblk = pl.BlockSpec((D, D), lambda i: (0, 0))
    bblk = pl.BlockSpec((1, D), lambda i: (0, 0))
    outf = jax.ShapeDtypeStruct((N, D), jnp.float32)
    return pl.pallas_call(
        body,
        grid=(N // NB,),
        in_specs=[blk, wblk, wblk, wblk, bblk, bblk],
        out_specs=(blk, blk, blk),
        out_shape=(outf, outf, outf),
    )(x, w1a, w1b, na, eb1, nb1)


def _gather(xa, xb, row4, col4, idmat, c):
    @functools.partial(
        pl.kernel,
        out_type=jax.ShapeDtypeStruct((EC, D), jnp.float32),
        mesh=_mesh,
        scratch_types=[
            pltpu.VMEM((NJC, W), jnp.int32),
            pltpu.VMEM((NJC, W), jnp.int32),
            pltpu.VMEM((1, W), jnp.int32),
            pltpu.VMEM((W, D), jnp.float32),
            pltpu.VMEM((W, D), jnp.float32),
            pltpu.VMEM((W, D), jnp.float32),
            pltpu.VMEM((W, D), jnp.float32),
            pltpu.VMEM_SHARED((NS * W, D), jnp.float32),
            pltpu.VMEM_SHARED((NS * W, D), jnp.float32),
            pltpu.SemaphoreType.DMA,
            pltpu.SemaphoreType.DMA,
            pltpu.SemaphoreType.DMA,
            pltpu.SemaphoreType.DMA,
        ],
    )
    def k(xa_hbm, xb_hbm, row_hbm, col_hbm, idm_hbm, g_hbm,
          ridx, cidx, ida, r1a, r2a, r1b, r2b, spa, spb, sa, sb, swa, swb):
        cid = lax.axis_index("c")
        sid = lax.axis_index("s")
        wid = sid * NC + cid
        w0, cnt = _wrange(wid)
        my0 = sid * W

        pltpu.sync_copy(row_hbm.at[c, wid], ridx)
        pltpu.sync_copy(col_hbm.at[c, wid], cidx)
        pltpu.sync_copy(idm_hbm.at[sid], ida)

        pltpu.async_copy(xa_hbm.at[ridx.at[0]], r1a, sa)
        pltpu.async_copy(xb_hbm.at[cidx.at[0]], r2a, sa)

        def slot(k_, my1, my2, mysem, ot1, ot2, otsem, mysp, mywsem):
            @pl.when(k_ < cnt)
            def _():
                pltpu.make_async_copy(xa_hbm.at[ridx.at[0]], my1, mysem).wait()
                pltpu.make_async_copy(xb_hbm.at[cidx.at[0]], my2, mysem).wait()

                @pl.when(k_ + 1 < cnt)
                def _():
                    pltpu.async_copy(xa_hbm.at[ridx.at[k_ + 1]], ot1, otsem)
                    pltpu.async_copy(xb_hbm.at[cidx.at[k_ + 1]], ot2, otsem)

                @pl.when(k_ >= 2)
                def _():
                    pltpu.make_async_copy(mysp.at[pl.ds(my0, W)],
                                          g_hbm.at[pl.ds(0, W)], mywsem).wait()

                pltpu.sync_copy(my1, mysp.at[pl.ds(my0, W)])
                pltpu.sync_copy(my2, mysp.at[ida.at[0]], add=True)
                pltpu.async_copy(mysp.at[pl.ds(my0, W)],
                                 g_hbm.at[pl.ds((w0 + k_) * W, W)], mywsem)

        @pl.loop(0, NJC + 1, step=2)
        def _(k_):
            slot(k_, r1a, r2a, sa, r1b, r2b, sb, spa, swa)
            slot(k_ + 1, r1b, r2b, sb, r1a, r2a, sa, spb, swb)

        pltpu.make_async_copy(spa.at[pl.ds(my0, W)],
                              g_hbm.at[pl.ds(0, W)], swa).wait()
        pltpu.make_async_copy(spb.at[pl.ds(my0, W)],
                              g_hbm.at[pl.ds(0, W)], swb).wait()

    return k(xa, xb, row4, col4, idmat)


def _edge(g, ea_full, w1c, w2, eb2, c):
    EB = 3200
    NBLK = EC // EB

    def body(g_r, ea_r, w1_r, w2_r, b2_r, out_r):
        eav = ea_r[...]
        cc = jnp.dot(eav.astype(jnp.bfloat16), w1_r[...], preferred_element_type=jnp.float32)
        h = jnp.maximum(g_r[...] + cc, 0.0)
        out_r[...] = (eav + jnp.dot(h.astype(jnp.bfloat16), w2_r[...],
                                    preferred_element_type=jnp.float32) + b2_r[...])

    blk = pl.BlockSpec((EB, D), lambda i: (i, 0))
    eablk = pl.BlockSpec((EB, D), lambda i, c=c: (c * NBLK + i, 0))
    wblk = pl.BlockSpec((D, D), lambda i: (0, 0))
    bblk = pl.BlockSpec((1, D), lambda i: (0, 0))
    return pl.pallas_call(
        body,
        grid=(NBLK,),
        in_specs=[blk, eablk, wblk, wblk, bblk],
        out_specs=blk,
        out_shape=jax.ShapeDtypeStruct((EC, D), jnp.float32),
    )(g, ea_full, w1c, w2, eb2)


NJS = NWINC // 8 + 1


def _swrange(u):
    lw0 = u * (NWINC // 8) + jnp.minimum(u, NWINC % 8)
    lcnt = NWINC // 8 + (u < NWINC % 8).astype(jnp.int32)
    return lw0, lcnt


def _scatter(eouts, col3, zrows):
    @functools.partial(
        pl.kernel,
        out_type=(jax.ShapeDtypeStruct((NC, NPAD, D), jnp.float32),
                  jax.ShapeDtypeStruct((E, D), jnp.float32)),
        mesh=_mesh,
        scratch_types=[
            pltpu.VMEM((NJS, W), jnp.int32),
            pltpu.VMEM((W, D), jnp.float32),
            pltpu.VMEM((W, D), jnp.float32),
            pltpu.VMEM_SHARED((NPAD, D), jnp.float32),
            pltpu.SemaphoreType.DMA,
            pltpu.SemaphoreType.DMA,
            pltpu.SemaphoreType.DMA,
            pltpu.SemaphoreType.DMA,
        ],
    )
    def k(e0_hbm, e1_hbm, e2_hbm, e3_hbm, col_hbm, z_hbm, out_hbm, eo_hbm,
          cidx, ebufa, ebufb, acc, sa, sb, swa, swb):
        cid = lax.axis_index("c")
        sid = lax.axis_index("s")
        wid = sid * NC + cid
        u = lax.rem(wid, 8)
        lw0, lcnt = _swrange(u)

        pltpu.sync_copy(z_hbm, acc.at[pl.ds(sid * ZR, ZR)])
        pltpu.sync_copy(col_hbm.at[wid], cidx)
        plsc.subcore_barrier()

        def chunk_loop(e_hbm, c):
            pltpu.async_copy(e_hbm.at[pl.ds(lw0 * W, W)], ebufa, sa)

            def slot(k_, mybuf, mysem, otbuf, otsem, otwsem, mywsem):
                @pl.when(k_ < lcnt)
                def _():
                    pltpu.make_async_copy(e_hbm.at[pl.ds(0, W)], mybuf,
                                          mysem).wait()

                    @pl.when(k_ + 1 < lcnt)
                    def _():
                        @pl.when(k_ >= 1)
                        def _():
                            pltpu.make_async_copy(
                                otbuf, eo_hbm.at[pl.ds(0, W)], otwsem).wait()

                        pltpu.async_copy(e_hbm.at[pl.ds((lw0 + k_ + 1) * W, W)],
                                         otbuf, otsem)

                    pltpu.sync_copy(mybuf, acc.at[cidx.at[k_]], add=True)
                    pltpu.async_copy(
                        mybuf, eo_hbm.at[pl.ds(c * EC + (lw0 + k_) * W, W)],
                        mywsem)

            @pl.loop(0, NJS + 1, step=2)
            def _(k_):
                slot(k_, ebufa, sa, ebufb, sb, swb, swa)
                slot(k_ + 1, ebufb, sb, ebufa, sa, swa, swb)

            pltpu.make_async_copy(ebufa, eo_hbm.at[pl.ds(0, W)], swa).wait()
            pltpu.make_async_copy(ebufb, eo_hbm.at[pl.ds(0, W)], swb).wait()

        for c, e_hbm in enumerate((e0_hbm, e1_hbm, e2_hbm, e3_hbm)):
            @pl.when(jnp.logical_and(wid >= 8 * c, wid < 8 * (c + 1)))
            def _(e_hbm=e_hbm, c=c):
                chunk_loop(e_hbm, c)

        plsc.subcore_barrier()
        pltpu.sync_copy(acc.at[pl.ds(sid * ZR, ZR)],
                        out_hbm.at[cid, pl.ds(sid * ZR, ZR)])

    return k(*eouts, col3, zrows)


def _node(x, xn, partials, nbw, nw2, nb2):
    NB = 2000

    def body(x_r, xn_r, p_r, nb_r, w2_r, b2_r, out_r):
        agg = p_r[0] + p_r[1]
        h2 = jnp.maximum(xn_r[...] + jnp.dot(agg, nb_r[...], preferred_element_type=jnp.float32), 0.0)
        out_r[...] = x_r[...] + jnp.dot(h2, w2_r[...], preferred_element_type=jnp.float32) + b2_r[...]

    blk = pl.BlockSpec((NB, D), lambda i: (i, 0))
    pblk = pl.BlockSpec((NC, NB, D), lambda i: (0, i, 0))
    wblk = pl.BlockSpec((D, D), lambda i: (0, 0))
    bblk = pl.BlockSpec((1, D), lambda i: (0, 0))
    return pl.pallas_call(
        body,
        grid=(N // NB,),
        in_specs=[blk, blk, pblk, wblk, wblk, bblk],
        out_specs=blk,
        out_shape=jax.ShapeDtypeStruct((N, D), jnp.float32),
    )(x, xn, partials, nbw, nw2, nb2)


def kernel(x, edge_attr, edge_index, eW1, eb1, eW2, eb2, nW1, nb1, nW2, nb2):
    w1a, w1b, w1c = eW1[:D], eW1[D:2 * D], eW1[2 * D:]
    na, nbw = nW1[:D], nW1[D:]
    eb1r = eb1.reshape(1, D)
    eb2r = eb2.reshape(1, D)
    nb1r = nb1.reshape(1, D)
    nb2r = nb2.reshape(1, D)

    row = edge_index[0].astype(jnp.int32)
    col = edge_index[1].astype(jnp.int32)
    row2 = jnp.pad(row, (0, W)).reshape(NWIN + 1, W)
    col2 = jnp.pad(col, (0, W)).reshape(NWIN + 1, W)
    gw = np.array([[c * NWINC + t * (NWINC // NWK) + min(t, NWINC % NWK) + j
                    for t in range(NWK) for j in range(NJC)]
                   for c in range(NCHUNK)]).reshape(-1)
    row4 = jnp.take(row2, gw, axis=0).reshape(NCHUNK, NWK, NJC, W)
    col4 = jnp.take(col2, gw, axis=0).reshape(NCHUNK, NWK, NJC, W)
    sw = np.array([(t // 8) * NWINC + (t % 8) * (NWINC // 8)
                   + min(t % 8, NWINC % 8) + j
                   for t in range(NWK) for j in range(NJS)])
    col3s = jnp.take(col2, sw, axis=0).reshape(NWK, NJS, W)

    idmat = (jnp.arange(NS, dtype=jnp.int32)[:, None, None] * W
             + jnp.arange(W, dtype=jnp.int32)[None, None, :])
    zrows = jnp.zeros((ZR, D), jnp.float32)
    w1cb = w1c.astype(jnp.bfloat16)
    w2b = eW2.astype(jnp.bfloat16)

    xa, xb, xn = _pre(x, w1a, w1b, na, eb1r, nb1r)

    eouts = []
    for c in range(NCHUNK):
        g_c = _gather(xa, xb, row4, col4, idmat, c)
        eouts.append(_edge(g_c, edge_attr, w1cb, w2b, eb2r, c))

    partials, eout = _scatter(eouts, col3s, zrows)
    x_new = _node(x, xn, partials, nbw, nW2, nb2r)
    return (x_new, eout)


# speedup vs baseline: 1.1351x; 1.0174x over previous
import functools

import jax
import jax.numpy as jnp
import numpy as np
from jax import lax
from jax.experimental import pallas as pl
from jax.experimental.pallas import tpu as pltpu
from jax.experimental.pallas import tpu_sc as plsc

N = 10000
E = 320000
D = 128
NPAD = 10240
W = 128
NWIN = E // W
NC = 2
NS = 16
NWK = NC * NS
ZR = NPAD // NS

NCHUNK = 4
EC = E // NCHUNK
NWINC = NWIN // NCHUNK
NJC = NWINC // NWK + 1

_mesh = plsc.VectorSubcoreMesh(core_axis_name="c", subcore_axis_name="s")


def _wrange(wid):
    w0 = wid * (NWINC // NWK) + jnp.minimum(wid, NWINC % NWK)
    cnt = NWINC // NWK + (wid < NWINC % NWK).astype(jnp.int32)
    return w0, cnt


def _pre(x, w1a, w1b, na, eb1, nb1):
    def body(x_r, wa_r, wb_r, na_r, eb1_r, nb1_r, xa_r, xb_r, xn_r):
        xv = x_r[...]
        xa_r[...] = jnp.dot(xv, wa_r[...], preferred_element_type=jnp.float32) + eb1_r[...]
        xb_r[...] = jnp.dot(xv, wb_r[...], preferred_element_type=jnp.float32)
        xn_r[...] = jnp.dot(xv, na_r[...], preferred_element_type=jnp.float32) + nb1_r[...]

    NB = 2000
    blk = pl.BlockSpec((NB, D), lambda i: (i, 0))
    wblk = pl.BlockSpec((D, D), lambda i: (0, 0))
    bblk = pl.BlockSpec((1, D), lambda i: (0, 0))
    outf = jax.ShapeDtypeStruct((N, D), jnp.float32)
    return pl.pallas_call(
        body,
        grid=(N // NB,),
        in_specs=[blk, wblk, wblk, wblk, bblk, bblk],
        out_specs=(blk, blk, blk),
        out_shape=(outf, outf, outf),
    )(x, w1a, w1b, na, eb1, nb1)


def _gather(xa, xb, row4, col4, idmat, c):
    @functools.partial(
        pl.kernel,
        out_type=jax.ShapeDtypeStruct((EC, D), jnp.float32),
        mesh=_mesh,
        scratch_types=[
            pltpu.VMEM((NJC, W), jnp.int32),
            pltpu.VMEM((NJC, W), jnp.int32),
            pltpu.VMEM((1, W), jnp.int32),
            pltpu.VMEM((W, D), jnp.float32),
            pltpu.VMEM((W, D), jnp.float32),
            pltpu.VMEM((W, D), jnp.float32),
            pltpu.VMEM((W, D), jnp.float32),
            pltpu.VMEM_SHARED((NS * W, D), jnp.float32),
            pltpu.VMEM_SHARED((NS * W, D), jnp.float32),
            pltpu.SemaphoreType.DMA,
            pltpu.SemaphoreType.DMA,
            pltpu.SemaphoreType.DMA,
            pltpu.SemaphoreType.DMA,
        ],
    )
    def k(xa_hbm, xb_hbm, row_hbm, col_hbm, idm_hbm, g_hbm,
          ridx, cidx, ida, r1a, r2a, r1b, r2b, spa, spb, sa, sb, swa, swb):
        cid = lax.axis_index("c")
        sid = lax.axis_index("s")
        wid = sid * NC + cid
        w0, cnt = _wrange(wid)
        my0 = sid * W

        pltpu.sync_copy(row_hbm.at[c, wid], ridx)
        pltpu.sync_copy(col_hbm.at[c, wid], cidx)
        pltpu.sync_copy(idm_hbm.at[sid], ida)

        pltpu.async_copy(xa_hbm.at[ridx.at[0]], r1a, sa)
        pltpu.async_copy(xb_hbm.at[cidx.at[0]], r2a, sa)

        def slot(k_, my1, my2, mysem, ot1, ot2, otsem, mysp, mywsem):
            @pl.when(k_ < cnt)
            def _():
                pltpu.make_async_copy(xa_hbm.at[ridx.at[0]], my1, mysem).wait()
                pltpu.make_async_copy(xb_hbm.at[cidx.at[0]], my2, mysem).wait()

                @pl.when(k_ + 1 < cnt)
                def _():
                    pltpu.async_copy(xa_hbm.at[ridx.at[k_ + 1]], ot1, otsem)
                    pltpu.async_copy(xb_hbm.at[cidx.at[k_ + 1]], ot2, otsem)

                @pl.when(k_ >= 2)
                def _():
                    pltpu.make_async_copy(mysp.at[pl.ds(my0, W)],
                                          g_hbm.at[pl.ds(0, W)], mywsem).wait()

                pltpu.sync_copy(my1, mysp.at[pl.ds(my0, W)])
                pltpu.sync_copy(my2, mysp.at[ida.at[0]], add=True)
                pltpu.async_copy(mysp.at[pl.ds(my0, W)],
                                 g_hbm.at[pl.ds((w0 + k_) * W, W)], mywsem)

        @pl.loop(0, NJC + 1, step=2)
        def _(k_):
            slot(k_, r1a, r2a, sa, r1b, r2b, sb, spa, swa)
            slot(k_ + 1, r1b, r2b, sb, r1a, r2a, sa, spb, swb)

        pltpu.make_async_copy(spa.at[pl.ds(my0, W)],
                              g_hbm.at[pl.ds(0, W)], swa).wait()
        pltpu.make_async_copy(spb.at[pl.ds(my0, W)],
                              g_hbm.at[pl.ds(0, W)], swb).wait()

    return k(xa, xb, row4, col4, idmat)


def _edge(g, ea_full, w1c, w2, eb2, c):
    EB = 3200
    NBLK = EC // EB

    def body(g_r, ea_r, w1_r, w2_r, b2_r, out_r):
        eav = ea_r[...]
        cc = jnp.dot(eav.astype(jnp.bfloat16), w1_r[...], preferred_element_type=jnp.float32)
        h = jnp.maximum(g_r[...] + cc, 0.0)
        out_r[...] = (eav + jnp.dot(h.astype(jnp.bfloat16), w2_r[...],
                                    preferred_element_type=jnp.float32) + b2_r[...])

    blk = pl.BlockSpec((EB, D), lambda i: (i, 0))
    eablk = pl.BlockSpec((EB, D), lambda i, c=c: (c * NBLK + i, 0))
    wblk = pl.BlockSpec((D, D), lambda i: (0, 0))
    bblk = pl.BlockSpec((1, D), lambda i: (0, 0))
    return pl.pallas_call(
        body,
        grid=(NBLK,),
        in_specs=[blk, eablk, wblk, wblk, bblk],
        out_specs=blk,
        out_shape=jax.ShapeDtypeStruct((EC, D), jnp.float32),
    )(g, ea_full, w1c, w2, eb2)


NJS = NWINC // 8 + 1


def _swrange(u):
    lw0 = u * (NWINC // 8) + jnp.minimum(u, NWINC % 8)
    lcnt = NWINC // 8 + (u < NWINC % 8).astype(jnp.int32)
    return lw0, lcnt


def _scatter(eouts, col3, zrows):
    @functools.partial(
        pl.kernel,
        out_type=(jax.ShapeDtypeStruct((NC, NPAD, D), jnp.float32),
                  jax.ShapeDtypeStruct((E, D), jnp.float32)),
        mesh=_mesh,
        scratch_types=[
            pltpu.VMEM((NJS, W), jnp.int32),
            pltpu.VMEM((W, D), jnp.float32),
            pltpu.VMEM((W, D), jnp.float32),
            pltpu.VMEM_SHARED((NPAD, D), jnp.float32),
            pltpu.SemaphoreType.DMA,
            pltpu.SemaphoreType.DMA,
            pltpu.SemaphoreType.DMA,
            pltpu.SemaphoreType.DMA,
        ],
    )
    def k(e0_hbm, e1_hbm, e2_hbm, e3_hbm, col_hbm, z_hbm, out_hbm, eo_hbm,
          cidx, ebufa, ebufb, acc, sa, sb, swa, swb):
        cid = lax.axis_index("c")
        sid = lax.axis_index("s")
        wid = sid * NC + cid
        u = lax.rem(wid, 8)
        lw0, lcnt = _swrange(u)

        pltpu.sync_copy(z_hbm, acc.at[pl.ds(sid * ZR, ZR)])
        pltpu.sync_copy(col_hbm.at[wid], cidx)
        plsc.subcore_barrier()

        def chunk_loop(e_hbm, c):
            pltpu.async_copy(e_hbm.at[pl.ds(lw0 * W, W)], ebufa, sa)

            def slot(k_, mybuf, mysem, otbuf, otsem, otwsem, mywsem):
                @pl.when(k_ < lcnt)
                def _():
                    pltpu.make_async_copy(e_hbm.at[pl.ds(0, W)], mybuf,
                                          mysem).wait()

                    @pl.when(k_ + 1 < lcnt)
                    def _():
                        @pl.when(k_ >= 1)
                        def _():
                            pltpu.make_async_copy(
                                otbuf, eo_hbm.at[pl.ds(0, W)], otwsem).wait()

                        pltpu.async_copy(e_hbm.at[pl.ds((lw0 + k_ + 1) * W, W)],
                                         otbuf, otsem)

                    pltpu.sync_copy(mybuf, acc.at[cidx.at[k_]], add=True)
                    pltpu.async_copy(
                        mybuf, eo_hbm.at[pl.ds(c * EC + (lw0 + k_) * W, W)],
                        mywsem)

            @pl.loop(0, NJS + 1, step=2)
            def _(k_):
                slot(k_, ebufa, sa, ebufb, sb, swb, swa)
                slot(k_ + 1, ebufb, sb, ebufa, sa, swa, swb)

            pltpu.make_async_copy(ebufa, eo_hbm.at[pl.ds(0, W)], swa).wait()
            pltpu.make_async_copy(ebufb, eo_hbm.at[pl.ds(0, W)], swb).wait()

        for c, e_hbm in enumerate((e0_hbm, e1_hbm, e2_hbm, e3_hbm)):
            @pl.when(jnp.logical_and(wid >= 8 * c, wid < 8 * (c + 1)))
            def _(e_hbm=e_hbm, c=c):
                chunk_loop(e_hbm, c)

        plsc.subcore_barrier()
        pltpu.sync_copy(acc.at[pl.ds(sid * ZR, ZR)],
                        out_hbm.at[cid, pl.ds(sid * ZR, ZR)])

    return k(*eouts, col3, zrows)


def _node(x, xn, partials, nbw, nw2, nb2):
    NB = 2000

    def body(x_r, xn_r, p_r, nb_r, w2_r, b2_r, out_r):
        agg = p_r[0] + p_r[1]
        h2 = jnp.maximum(xn_r[...] + jnp.dot(agg, nb_r[...], preferred_element_type=jnp.float32), 0.0)
        out_r[...] = x_r[...] + jnp.dot(h2, w2_r[...], preferred_element_type=jnp.float32) + b2_r[...]

    blk = pl.BlockSpec((NB, D), lambda i: (i, 0))
    pblk = pl.BlockSpec((NC, NB, D), lambda i: (0, i, 0))
    wblk = pl.BlockSpec((D, D), lambda i: (0, 0))
    bblk = pl.BlockSpec((1, D), lambda i: (0, 0))
    return pl.pallas_call(
        body,
        grid=(N // NB,),
        in_specs=[blk, blk, pblk, wblk, wblk, bblk],
        out_specs=blk,
        out_shape=jax.ShapeDtypeStruct((N, D), jnp.float32),
    )(x, xn, partials, nbw, nw2, nb2)


def kernel(x, edge_attr, edge_index, eW1, eb1, eW2, eb2, nW1, nb1, nW2, nb2):
    w1a, w1b, w1c = eW1[:D], eW1[D:2 * D], eW1[2 * D:]
    na, nbw = nW1[:D], nW1[D:]
    eb1r = eb1.reshape(1, D)
    eb2r = eb2.reshape(1, D)
    nb1r = nb1.reshape(1, D)
    nb2r = nb2.reshape(1, D)

    ei2 = edge_index.astype(jnp.int32).reshape(2 * NWIN, W)

    def wsafe(w):
        return w if w < NWIN else 0

    gw = np.array([[wsafe(c * NWINC + t * (NWINC // NWK) + min(t, NWINC % NWK) + j)
                    for t in range(NWK) for j in range(NJC)]
                   for c in range(NCHUNK)]).reshape(-1)
    row4 = jnp.take(ei2, gw, axis=0).reshape(NCHUNK, NWK, NJC, W)
    col4 = jnp.take(ei2, gw + NWIN, axis=0).reshape(NCHUNK, NWK, NJC, W)
    sw = np.array([wsafe((t // 8) * NWINC + (t % 8) * (NWINC // 8)
                         + min(t % 8, NWINC % 8) + j)
                   for t in range(NWK) for j in range(NJS)])
    col3s = jnp.take(ei2, sw + NWIN, axis=0).reshape(NWK, NJS, W)

    idmat = (jnp.arange(NS, dtype=jnp.int32)[:, None, None] * W
             + jnp.arange(W, dtype=jnp.int32)[None, None, :])
    zrows = jnp.zeros((ZR, D), jnp.float32)
    w1cb = w1c.astype(jnp.bfloat16)
    w2b = eW2.astype(jnp.bfloat16)

    xa, xb, xn = _pre(x, w1a, w1b, na, eb1r, nb1r)

    eouts = []
    for c in range(NCHUNK):
        g_c = _gather(xa, xb, row4, col4, idmat, c)
        eouts.append(_edge(g_c, edge_attr, w1cb, w2b, eb2r, c))

    partials, eout = _scatter(eouts, col3s, zrows)
    x_new = _node(x, xn, partials, nbw, nW2, nb2r)
    return (x_new, eout)


# speedup vs baseline: 1.1396x; 1.0040x over previous
import functools

import jax
import jax.numpy as jnp
import numpy as np
from jax import lax
from jax.experimental import pallas as pl
from jax.experimental.pallas import tpu as pltpu
from jax.experimental.pallas import tpu_sc as plsc

N = 10000
E = 320000
D = 128
NPAD = 10240
W = 128
NWIN = E // W
NC = 2
NS = 16
NWK = NC * NS
ZR = NPAD // NS

NCHUNK = 4
EC = E // NCHUNK
NWINC = NWIN // NCHUNK
NJC = NWINC // NWK + 1

_mesh = plsc.VectorSubcoreMesh(core_axis_name="c", subcore_axis_name="s")


def _wrange(wid):
    w0 = wid * (NWINC // NWK) + jnp.minimum(wid, NWINC % NWK)
    cnt = NWINC // NWK + (wid < NWINC % NWK).astype(jnp.int32)
    return w0, cnt


def _pre(x, w1a, w1b, na, eb1, nb1):
    def body(x_r, wa_r, wb_r, na_r, eb1_r, nb1_r, xa_r, xb_r, xn_r):
        xv = x_r[...]
        xa_r[...] = jnp.dot(xv, wa_r[...], preferred_element_type=jnp.float32) + eb1_r[...]
        xb_r[...] = jnp.dot(xv, wb_r[...], preferred_element_type=jnp.float32)
        xn_r[...] = jnp.dot(xv, na_r[...], preferred_element_type=jnp.float32) + nb1_r[...]

    NB = 2000
    blk = pl.BlockSpec((NB, D), lambda i: (i, 0))
    wblk = pl.BlockSpec((D, D), lambda i: (0, 0))
    bblk = pl.BlockSpec((1, D), lambda i: (0, 0))
    outf = jax.ShapeDtypeStruct((N, D), jnp.float32)
    return pl.pallas_call(
        body,
        grid=(N // NB,),
        in_specs=[blk, wblk, wblk, wblk, bblk, bblk],
        out_specs=(blk, blk, blk),
        out_shape=(outf, outf, outf),
    )(x, w1a, w1b, na, eb1, nb1)


def _gather(xa, xb, row4, col4, idmat, c):
    @functools.partial(
        pl.kernel,
        out_type=jax.ShapeDtypeStruct((EC, D), jnp.float32),
        mesh=_mesh,
        scratch_types=[
            pltpu.VMEM((NJC, W), jnp.int32),
            pltpu.VMEM((NJC, W), jnp.int32),
            pltpu.VMEM((1, W), jnp.int32),
            pltpu.VMEM((W, D), jnp.float32),
            pltpu.VMEM((W, D), jnp.float32),
            pltpu.VMEM((W, D), jnp.float32),
            pltpu.VMEM((W, D), jnp.float32),
            pltpu.VMEM_SHARED((NS * W, D), jnp.float32),
            pltpu.VMEM_SHARED((NS * W, D), jnp.float32),
            pltpu.SemaphoreType.DMA,
            pltpu.SemaphoreType.DMA,
            pltpu.SemaphoreType.DMA,
            pltpu.SemaphoreType.DMA,
        ],
    )
    def k(xa_hbm, xb_hbm, row_hbm, col_hbm, idm_hbm, g_hbm,
          ridx, cidx, ida, r1a, r2a, r1b, r2b, spa, spb, sa, sb, swa, swb):
        cid = lax.axis_index("c")
        sid = lax.axis_index("s")
        wid = sid * NC + cid
        w0, cnt = _wrange(wid)
        my0 = sid * W

        pltpu.sync_copy(row_hbm.at[c, wid], ridx)
        pltpu.sync_copy(col_hbm.at[c, wid], cidx)
        pltpu.sync_copy(idm_hbm.at[sid], ida)

        pltpu.async_copy(xa_hbm.at[ridx.at[0]], r1a, sa)
        pltpu.async_copy(xb_hbm.at[cidx.at[0]], r2a, sa)

        def slot(k_, my1, my2, mysem, ot1, ot2, otsem, mysp, mywsem):
            @pl.when(k_ < cnt)
            def _():
                pltpu.make_async_copy(xa_hbm.at[ridx.at[0]], my1, mysem).wait()
                pltpu.make_async_copy(xb_hbm.at[cidx.at[0]], my2, mysem).wait()

                @pl.when(k_ + 1 < cnt)
                def _():
                    pltpu.async_copy(xa_hbm.at[ridx.at[k_ + 1]], ot1, otsem)
                    pltpu.async_copy(xb_hbm.at[cidx.at[k_ + 1]], ot2, otsem)

                @pl.when(k_ >= 2)
                def _():
                    pltpu.make_async_copy(mysp.at[pl.ds(my0, W)],
                                          g_hbm.at[pl.ds(0, W)], mywsem).wait()

                pltpu.sync_copy(my1, mysp.at[pl.ds(my0, W)])
                pltpu.sync_copy(my2, mysp.at[ida.at[0]], add=True)
                pltpu.async_copy(mysp.at[pl.ds(my0, W)],
                                 g_hbm.at[pl.ds((w0 + k_) * W, W)], mywsem)

        @pl.loop(0, NJC + 1, step=2)
        def _(k_):
            slot(k_, r1a, r2a, sa, r1b, r2b, sb, spa, swa)
            slot(k_ + 1, r1b, r2b, sb, r1a, r2a, sa, spb, swb)

        pltpu.make_async_copy(spa.at[pl.ds(my0, W)],
                              g_hbm.at[pl.ds(0, W)], swa).wait()
        pltpu.make_async_copy(spb.at[pl.ds(my0, W)],
                              g_hbm.at[pl.ds(0, W)], swb).wait()

    return k(xa, xb, row4, col4, idmat)


def _edge(g, ea_full, w1c, w2, eb2, c):
    EB = 4000
    NBLK = EC // EB

    def body(g_r, ea_r, w1_r, w2_r, b2_r, out_r):
        eav = ea_r[...]
        cc = jnp.dot(eav.astype(jnp.bfloat16), w1_r[...], preferred_element_type=jnp.float32)
        h = jnp.maximum(g_r[...] + cc, 0.0)
        out_r[...] = (eav + jnp.dot(h.astype(jnp.bfloat16), w2_r[...],
                                    preferred_element_type=jnp.float32) + b2_r[...])

    blk = pl.BlockSpec((EB, D), lambda i: (i, 0))
    eablk = pl.BlockSpec((EB, D), lambda i, c=c: (c * NBLK + i, 0))
    wblk = pl.BlockSpec((D, D), lambda i: (0, 0))
    bblk = pl.BlockSpec((1, D), lambda i: (0, 0))
    return pl.pallas_call(
        body,
        grid=(NBLK,),
        in_specs=[blk, eablk, wblk, wblk, bblk],
        out_specs=blk,
        out_shape=jax.ShapeDtypeStruct((EC, D), jnp.float32),
    )(g, ea_full, w1c, w2, eb2)


NJS = NWINC // 8 + 1


def _swrange(u):
    lw0 = u * (NWINC // 8) + jnp.minimum(u, NWINC % 8)
    lcnt = NWINC // 8 + (u < NWINC % 8).astype(jnp.int32)
    return lw0, lcnt


def _scatter(eouts, col3, zrows):
    @functools.partial(
        pl.kernel,
        out_type=(jax.ShapeDtypeStruct((NC, NPAD, D), jnp.float32),
                  jax.ShapeDtypeStruct((E, D), jnp.float32)),
        mesh=_mesh,
        scratch_types=[
            pltpu.VMEM((NJS, W), jnp.int32),
            pltpu.VMEM((W, D), jnp.float32),
            pltpu.VMEM((W, D), jnp.float32),
            pltpu.VMEM_SHARED((NPAD, D), jnp.float32),
            pltpu.SemaphoreType.DMA,
            pltpu.SemaphoreType.DMA,
            pltpu.SemaphoreType.DMA,
            pltpu.SemaphoreType.DMA,
        ],
    )
    def k(e0_hbm, e1_hbm, e2_hbm, e3_hbm, col_hbm, z_hbm, out_hbm, eo_hbm,
          cidx, ebufa, ebufb, acc, sa, sb, swa, swb):
        cid = lax.axis_index("c")
        sid = lax.axis_index("s")
        wid = sid * NC + cid
        u = lax.rem(wid, 8)
        lw0, lcnt = _swrange(u)

        pltpu.sync_copy(z_hbm, acc.at[pl.ds(sid * ZR, ZR)])
        pltpu.sync_copy(col_hbm.at[wid], cidx)
        plsc.subcore_barrier()

        def chunk_loop(e_hbm, c):
            pltpu.async_copy(e_hbm.at[pl.ds(lw0 * W, W)], ebufa, sa)

            def slot(k_, mybuf, mysem, otbuf, otsem, otwsem, mywsem):
                @pl.when(k_ < lcnt)
                def _():
                    pltpu.make_async_copy(e_hbm.at[pl.ds(0, W)], mybuf,
                                          mysem).wait()

                    @pl.when(k_ + 1 < lcnt)
                    def _():
                        @pl.when(k_ >= 1)
                        def _():
                            pltpu.make_async_copy(
                                otbuf, eo_hbm.at[pl.ds(0, W)], otwsem).wait()

                        pltpu.async_copy(e_hbm.at[pl.ds((lw0 + k_ + 1) * W, W)],
                                         otbuf, otsem)

                    pltpu.sync_copy(mybuf, acc.at[cidx.at[k_]], add=True)
                    pltpu.async_copy(
                        mybuf, eo_hbm.at[pl.ds(c * EC + (lw0 + k_) * W, W)],
                        mywsem)

            @pl.loop(0, NJS + 1, step=2)
            def _(k_):
                slot(k_, ebufa, sa, ebufb, sb, swb, swa)
                slot(k_ + 1, ebufb, sb, ebufa, sa, swa, swb)

            pltpu.make_async_copy(ebufa, eo_hbm.at[pl.ds(0, W)], swa).wait()
            pltpu.make_async_copy(ebufb, eo_hbm.at[pl.ds(0, W)], swb).wait()

        for c, e_hbm in enumerate((e0_hbm, e1_hbm, e2_hbm, e3_hbm)):
            @pl.when(jnp.logical_and(wid >= 8 * c, wid < 8 * (c + 1)))
            def _(e_hbm=e_hbm, c=c):
                chunk_loop(e_hbm, c)

        plsc.subcore_barrier()
        pltpu.sync_copy(acc.at[pl.ds(sid * ZR, ZR)],
                        out_hbm.at[cid, pl.ds(sid * ZR, ZR)])

    return k(*eouts, col3, zrows)


def _node(x, xn, partials, nbw, nw2, nb2):
    NB = 2000

    def body(x_r, xn_r, p_r, nb_r, w2_r, b2_r, out_r):
        agg = p_r[0] + p_r[1]
        h2 = jnp.maximum(xn_r[...] + jnp.dot(agg, nb_r[...], preferred_element_type=jnp.float32), 0.0)
        out_r[...] = x_r[...] + jnp.dot(h2, w2_r[...], preferred_element_type=jnp.float32) + b2_r[...]

    blk = pl.BlockSpec((NB, D), lambda i: (i, 0))
    pblk = pl.BlockSpec((NC, NB, D), lambda i: (0, i, 0))
    wblk = pl.BlockSpec((D, D), lambda i: (0, 0))
    bblk = pl.BlockSpec((1, D), lambda i: (0, 0))
    return pl.pallas_call(
        body,
        grid=(N // NB,),
        in_specs=[blk, blk, pblk, wblk, wblk, bblk],
        out_specs=blk,
        out_shape=jax.ShapeDtypeStruct((N, D), jnp.float32),
    )(x, xn, partials, nbw, nw2, nb2)


def kernel(x, edge_attr, edge_index, eW1, eb1, eW2, eb2, nW1, nb1, nW2, nb2):
    w1a, w1b, w1c = eW1[:D], eW1[D:2 * D], eW1[2 * D:]
    na, nbw = nW1[:D], nW1[D:]
    eb1r = eb1.reshape(1, D)
    eb2r = eb2.reshape(1, D)
    nb1r = nb1.reshape(1, D)
    nb2r = nb2.reshape(1, D)

    ei2 = edge_index.astype(jnp.int32).reshape(2 * NWIN, W)

    def wsafe(w):
        return w if w < NWIN else 0

    gw = np.array([[wsafe(c * NWINC + t * (NWINC // NWK) + min(t, NWINC % NWK) + j)
                    for t in range(NWK) for j in range(NJC)]
                   for c in range(NCHUNK)]).reshape(-1)
    row4 = jnp.take(ei2, gw, axis=0).reshape(NCHUNK, NWK, NJC, W)
    col4 = jnp.take(ei2, gw + NWIN, axis=0).reshape(NCHUNK, NWK, NJC, W)
    sw = np.array([wsafe((t // 8) * NWINC + (t % 8) * (NWINC // 8)
                         + min(t % 8, NWINC % 8) + j)
                   for t in range(NWK) for j in range(NJS)])
    col3s = jnp.take(ei2, sw + NWIN, axis=0).reshape(NWK, NJS, W)

    idmat = (jnp.arange(NS, dtype=jnp.int32)[:, None, None] * W
             + jnp.arange(W, dtype=jnp.int32)[None, None, :])
    zrows = jnp.zeros((ZR, D), jnp.float32)
    w1cb = w1c.astype(jnp.bfloat16)
    w2b = eW2.astype(jnp.bfloat16)

    xa, xb, xn = _pre(x, w1a, w1b, na, eb1r, nb1r)

    eouts = []
    for c in range(NCHUNK):
        g_c = _gather(xa, xb, row4, col4, idmat, c)
        eouts.append(_edge(g_c, edge_attr, w1cb, w2b, eb2r, c))

    partials, eout = _scatter(eouts, col3s, zrows)
    x_new = _node(x, xn, partials, nbw, nW2, nb2r)
    return (x_new, eout)
